# Initial kernel scaffold; baseline (speedup 1.0000x reference)
import jax
import jax.numpy as jnp
from jax.experimental import pallas as pl


def kernel(edge_index, equi_edge_attr, inv_edge_attr, undirected_mask, W_ee, W_ei, W_ie, W_ii, W_fe, W_fi, b_fi):
    raise NotImplementedError("write your pallas kernel here")


# speedup vs baseline: 4.3450x; 4.3450x over previous
import jax
import jax.numpy as jnp
from jax import lax
from jax.experimental import pallas as pl
from jax.experimental.pallas import tpu as pltpu
from jax.experimental.pallas import tpu_sc as plsc

N = 50_000
E = 1_600_000
D = 16
NC, NS = 2, 16
NPADR = 50_176
DUMMY = N
RPT = NPADR // NS
ZR = 392

EPT_A = E // NS
EPT_C = E // (NC * NS)
GROUP = 2048
CH = 128
NFULLG_A = EPT_A // GROUP
TAILG_A = EPT_A - NFULLG_A * GROUP

_MESH = plsc.VectorSubcoreMesh(core_axis_name="c", subcore_axis_name="s")
_SC_PARAMS = pltpu.CompilerParams(use_tc_tiling_on_sc=False)


def _scatter_body(dst_hbm, mask_hbm, equi_hbm, inv_hbm, agg_hbm,
                  vals, dstb, maskb, idxb, zbuf, agg):
    c = lax.axis_index("c")
    s = lax.axis_index("s")

    def zb(i, _):
        zbuf[i] = jnp.zeros((D,), jnp.float32)
        return 0

    lax.fori_loop(0, ZR, zb, 0)
    for r in range(RPT // ZR):
        pltpu.sync_copy(zbuf, agg.at[pl.ds(s * RPT + r * ZR, ZR)])
    plsc.subcore_barrier()

    tile_base = s * EPT_A

    def process(attr_hbm, base, nedges):
        nfull, rem = divmod(nedges, CH)
        nchunks = nfull + (1 if rem else 0)
        pltpu.sync_copy(dst_hbm.at[pl.ds(base, nedges)], dstb.at[pl.ds(0, nedges)])
        pltpu.sync_copy(mask_hbm.at[pl.ds(base, nedges)], maskb.at[pl.ds(0, nedges)])
        pltpu.sync_copy(attr_hbm.at[pl.ds(base, nedges)], vals.at[pl.ds(0, nedges)])

        def row_body(j, _):
            for l in range(CH // 16):
                d = dstb[pl.ds(j * CH + l * 16, 16)]
                m = maskb[pl.ds(j * CH + l * 16, 16)]
                idxb[j, pl.ds(l * 16, 16)] = jnp.where(
                    m != 0, d, jnp.full((16,), DUMMY, jnp.int32))
            return 0

        lax.fori_loop(0, nfull, row_body, 0)
        if rem:
            for l in range(CH // 16):
                if (l + 1) * 16 <= rem:
                    d = dstb[pl.ds(nfull * CH + l * 16, 16)]
                    m = maskb[pl.ds(nfull * CH + l * 16, 16)]
                    idxb[nfull, pl.ds(l * 16, 16)] = jnp.where(
                        m != 0, d, jnp.full((16,), DUMMY, jnp.int32))
                else:
                    idxb[nfull, pl.ds(l * 16, 16)] = jnp.full((16,), DUMMY,
                                                              jnp.int32)
        for j in range(nchunks):
            pltpu.sync_copy(vals.at[pl.ds(j * CH, CH)], agg.at[idxb.at[j]],
                            add=True)

    def run(attr_hbm):
        def g_body(g, _):
            process(attr_hbm, tile_base + g * GROUP, GROUP)
            return 0

        lax.fori_loop(0, NFULLG_A, g_body, 0)
        process(attr_hbm, tile_base + NFULLG_A * GROUP, TAILG_A)

    @pl.when(c == 0)
    def _run_equi():
        run(equi_hbm)

    @pl.when(c != 0)
    def _run_inv():
        run(inv_hbm)

    plsc.subcore_barrier()
    pltpu.sync_copy(agg.at[pl.ds(s * RPT, RPT)],
                    agg_hbm.at[pl.ds(c * NPADR + s * RPT, RPT)])


_scatter_call = pl.kernel(
    _scatter_body,
    out_type=[jax.ShapeDtypeStruct((NC * NPADR, D), jnp.float32)],
    mesh=_MESH,
    scratch_types=[
        pltpu.VMEM((GROUP, D), jnp.float32),
        pltpu.VMEM((GROUP,), jnp.int32),
        pltpu.VMEM((GROUP,), jnp.int32),
        pltpu.VMEM((GROUP // CH, CH), jnp.int32),
        pltpu.VMEM((ZR, D), jnp.float32),
        pltpu.VMEM_SHARED((NPADR, D), jnp.float32),
    ],
    compiler_params=_SC_PARAMS,
)


def _gather_body(src_hbm, ne_hbm, ni_hbm, oe_hbm, oi_hbm,
                 idxf, rows_e, rows_i, sem):
    c = lax.axis_index("c")
    s = lax.axis_index("s")
    wid = s * NC + c
    tile_base = wid * EPT_C

    def process(base, nedges):
        nfull, rem = divmod(nedges, CH)
        nchunks = nfull + (1 if rem else 0)
        pltpu.sync_copy(src_hbm.at[pl.ds(base, nedges)], idxf.at[pl.ds(0, nedges)])
        descs = []
        for j in range(nchunks):
            nrows = CH if j < nfull else rem
            isl = idxf.at[pl.ds(j * CH, nrows)]
            descs.append(pltpu.async_copy(ne_hbm.at[isl],
                                          rows_e.at[pl.ds(j * CH, nrows)], sem))
            descs.append(pltpu.async_copy(ni_hbm.at[isl],
                                          rows_i.at[pl.ds(j * CH, nrows)], sem))
        for dsc in descs:
            dsc.wait()
        pltpu.sync_copy(rows_e.at[pl.ds(0, nedges)], oe_hbm.at[pl.ds(base, nedges)])
        pltpu.sync_copy(rows_i.at[pl.ds(0, nedges)], oi_hbm.at[pl.ds(base, nedges)])

    def g_body(g, _):
        process(tile_base + g * GROUP, GROUP)
        return 0

    lax.fori_loop(0, EPT_C // GROUP, g_body, 0)
    if EPT_C % GROUP:
        process(tile_base + (EPT_C // GROUP) * GROUP, EPT_C % GROUP)


_gather_call = pl.kernel(
    _gather_body,
    out_type=[jax.ShapeDtypeStruct((E, D), jnp.float32),
              jax.ShapeDtypeStruct((E, D), jnp.float32)],
    mesh=_MESH,
    scratch_types=[
        pltpu.VMEM((GROUP,), jnp.int32),
        pltpu.VMEM((GROUP, D), jnp.float32),
        pltpu.VMEM((GROUP, D), jnp.float32),
        pltpu.SemaphoreType.DMA,
    ],
    compiler_params=_SC_PARAMS,
)


BR = 3136


def _node_body(pe, pi, wee, wei, wie, wii, wfe, wfi, bfi, ne, ni):
    ae = pe[...]
    ai = pi[...]
    f32 = jnp.float32
    he = jnp.tanh(jnp.dot(ae, wee[...], preferred_element_type=f32)
                  + jnp.dot(ai, wie[...], preferred_element_type=f32))
    hi = jnp.maximum(jnp.dot(ai, wii[...], preferred_element_type=f32)
                     + jnp.dot(ae, wei[...], preferred_element_type=f32), 0.0)
    oe = (jnp.dot(he, wfe[0:D, :], preferred_element_type=f32)
          + jnp.dot(hi, wfe[D:2 * D, :], preferred_element_type=f32) + he)
    oi = (jnp.dot(jnp.abs(he), wfi[0:D, :], preferred_element_type=f32)
          + jnp.dot(hi, wfi[D:2 * D, :], preferred_element_type=f32)
          + bfi[...] + hi)
    ne[...] = oe
    ni[...] = oi


def _w_spec():
    return pl.BlockSpec((D, D), lambda i: (0, 0))


_node_call = pl.pallas_call(
    _node_body,
    grid=(NPADR // BR,),
    in_specs=[
        pl.BlockSpec((BR, D), lambda i: (i, 0)),
        pl.BlockSpec((BR, D), lambda i: (i + NPADR // BR, 0)),
        _w_spec(), _w_spec(), _w_spec(), _w_spec(),
        pl.BlockSpec((2 * D, D), lambda i: (0, 0)),
        pl.BlockSpec((2 * D, D), lambda i: (0, 0)),
        pl.BlockSpec((1, D), lambda i: (0, 0)),
    ],
    out_specs=[pl.BlockSpec((BR, D), lambda i: (i, 0)),
               pl.BlockSpec((BR, D), lambda i: (i, 0))],
    out_shape=[jax.ShapeDtypeStruct((NPADR, D), jnp.float32),
               jax.ShapeDtypeStruct((NPADR, D), jnp.float32)],
)


@jax.jit
def kernel(edge_index, equi_edge_attr, inv_edge_attr, undirected_mask,
           W_ee, W_ei, W_ie, W_ii, W_fe, W_fi, b_fi):
    src = edge_index[0]
    dst = edge_index[1]
    mask_i = undirected_mask.astype(jnp.int32)
    (agg,) = _scatter_call(dst, mask_i, equi_edge_attr, inv_edge_attr)
    node_e, node_i = _node_call(agg, agg, W_ee, W_ei, W_ie, W_ii,
                                W_fe, W_fi, b_fi.reshape(1, D))
    out_e, out_i = _gather_call(src, node_e, node_i)
    return out_e, out_i
